# baseline (speedup 1.0000x reference)
import functools
import math

import jax
import jax.numpy as jnp
import numpy as np
from jax import lax
from jax.experimental import pallas as pl
from jax.experimental.pallas import tpu as pltpu
from jax.experimental.pallas import tpu_sc as plsc

_NUM_BUCKETS = 32
_X_MAX_DISTANCE = 900
_Y_MAX_DISTANCE = 550
_SCALE = 0.125
_P = 900
_D = 2 * _P - 1
_DPAD = 1808

_NW = 32
_B = 2
_N = 2048
_ROWS = _B * _N
_RPW = _ROWS // _NW
_RB = 8
_NBLK = _RPW // _RB
_NCH = _N // 16


def _bucket_of_d(max_distance):
    d = np.arange(-(_P - 1), _P, dtype=np.int64)
    n = -d
    ret = (n < 0).astype(np.int32) * 16
    n = np.abs(n)
    max_exact = 8
    is_small = n < max_exact
    n8 = np.maximum(n, 1).astype(np.float32) / np.float32(max_exact)
    val_large = max_exact + (
        np.log(n8) / np.float32(math.log(max_distance / max_exact)) * np.float32(8)
    ).astype(np.int32)
    val_large = np.minimum(val_large, 15)
    return ret + np.where(is_small, n.astype(np.int32), val_large)

_BMAP_X = np.pad(_bucket_of_d(_X_MAX_DISTANCE), (0, _DPAD - _D)).astype(np.int32)
_BMAP_Y = np.pad(_bucket_of_d(_Y_MAX_DISTANCE), (0, _DPAD - _D)).astype(np.int32)


def _sc_body(qk, px, py, bmx, bmy, xt, yt, out,
             xcol, ycol, ftx, fty, tbl, bmbuf, buf0, buf1,
             isem0, isem1, osem0, osem1):
    wid = lax.axis_index("s") * 2 + lax.axis_index("c")
    b = wid // 16
    lrow0 = (wid % 16) * _RPW
    grow0 = b * _N + lrow0

    pltpu.sync_copy(px.at[b], xcol)
    pltpu.sync_copy(py.at[b], ycol)

    def _build(ft):
        def body(c, carry):
            s = pl.ds(c * 16, 16)
            v = plsc.load_gather(tbl, [bmbuf[s]])
            ft[s] = v * _SCALE
            return carry
        lax.fori_loop(0, _DPAD // 16, body, 0)

    pltpu.sync_copy(xt, tbl)
    pltpu.sync_copy(bmx, bmbuf)
    _build(ftx)
    pltpu.sync_copy(yt, tbl)
    pltpu.sync_copy(bmy, bmbuf)
    _build(fty)

    def _off(c, carry):
        s = pl.ds(c * 16, 16)
        xcol[s] = xcol[s] + (_P - 1)
        ycol[s] = ycol[s] + (_P - 1)
        return carry
    lax.fori_loop(0, _NCH, _off, 0)

    bufs = (buf0, buf1)
    isems = (isem0, isem1)
    osems = (osem0, osem1)

    def _in_copy(blk):
        return pltpu.make_async_copy(
            qk.at[pl.ds(grow0 + blk * _RB, _RB)], bufs[blk % 2], isems[blk % 2])

    def _out_copy(blk):
        return pltpu.make_async_copy(
            bufs[blk % 2], out.at[pl.ds(grow0 + blk * _RB, _RB)], osems[blk % 2])

    def _compute(buf, blk):
        lrow = lrow0 + blk * _RB
        lv = jnp.broadcast_to(lrow, (16,)).astype(jnp.int32)
        xis = [plsc.load_gather(xcol, [lv + r]) - (_P - 1) for r in range(_RB)]
        yis = [plsc.load_gather(ycol, [lv + r]) - (_P - 1) for r in range(_RB)]
        def col_body(c, carry2):
            s = pl.ds(c * 16, 16)
            xc = xcol[s]
            yc = ycol[s]
            for r in range(_RB):
                f = plsc.load_gather(ftx, [xc - xis[r]]) + plsc.load_gather(fty, [yc - yis[r]])
                buf[r, s] = buf[r, s] + f
            return carry2
        lax.fori_loop(0, _NCH, col_body, 0)

    _in_copy(0).start()
    for blk in range(_NBLK):
        _in_copy(blk).wait()
        if blk >= 1:
            _out_copy(blk - 1).wait()
        if blk + 1 < _NBLK:
            _in_copy(blk + 1).start()
        _compute(bufs[blk % 2], blk)
        _out_copy(blk).start()
    _out_copy(_NBLK - 1).wait()


_sc_bias = functools.partial(
    pl.kernel,
    mesh=plsc.VectorSubcoreMesh(core_axis_name="c", subcore_axis_name="s"),
    out_type=jax.ShapeDtypeStruct((_ROWS, _N), jnp.float32),
    scratch_types=[
        pltpu.VMEM((_N,), jnp.int32),
        pltpu.VMEM((_N,), jnp.int32),
        pltpu.VMEM((_DPAD,), jnp.float32),
        pltpu.VMEM((_DPAD,), jnp.float32),
        pltpu.VMEM((_NUM_BUCKETS,), jnp.float32),
        pltpu.VMEM((_DPAD,), jnp.int32),
        pltpu.VMEM((_RB, _N), jnp.float32),
        pltpu.VMEM((_RB, _N), jnp.float32),
        pltpu.SemaphoreType.DMA,
        pltpu.SemaphoreType.DMA,
        pltpu.SemaphoreType.DMA,
        pltpu.SemaphoreType.DMA,
    ],
    compiler_params=pltpu.CompilerParams(needs_layout_passes=False),
)(_sc_body)


def kernel(qk_dots, positions, x_table, y_table):
    b, n, _ = qk_dots.shape
    pos = positions.astype(jnp.int32)
    px = pos[:, :, 0]
    py = pos[:, :, 1]
    qk2 = qk_dots.reshape(b * n, n)
    out = _sc_bias(qk2, px, py,
                   jnp.asarray(_BMAP_X), jnp.asarray(_BMAP_Y),
                   x_table[:, 0], y_table[:, 0])
    return out.reshape(b, n, n)


# speedup vs baseline: 1.0924x; 1.0924x over previous
import functools
import math

import jax
import jax.numpy as jnp
import numpy as np
from jax import lax
from jax.experimental import pallas as pl
from jax.experimental.pallas import tpu as pltpu
from jax.experimental.pallas import tpu_sc as plsc

_NUM_BUCKETS = 32
_X_MAX_DISTANCE = 900
_Y_MAX_DISTANCE = 550
_SCALE = 0.125
_P = 900
_D = 2 * _P - 1
_DPAD = 1808

_NW = 32
_B = 2
_N = 2048
_ROWS = _B * _N
_RPW = _ROWS // _NW
_RB = 8
_NBLK = _RPW // _RB
_NCH = _N // 16


def _bucket_of_d(max_distance):
    d = np.arange(-(_P - 1), _P, dtype=np.int64)
    n = -d
    ret = (n < 0).astype(np.int32) * 16
    n = np.abs(n)
    max_exact = 8
    is_small = n < max_exact
    n8 = np.maximum(n, 1).astype(np.float32) / np.float32(max_exact)
    val_large = max_exact + (
        np.log(n8) / np.float32(math.log(max_distance / max_exact)) * np.float32(8)
    ).astype(np.int32)
    val_large = np.minimum(val_large, 15)
    return ret + np.where(is_small, n.astype(np.int32), val_large)

_BMAP_X = np.pad(_bucket_of_d(_X_MAX_DISTANCE), (0, _DPAD - _D)).astype(np.int32)
_BMAP_Y = np.pad(_bucket_of_d(_Y_MAX_DISTANCE), (0, _DPAD - _D)).astype(np.int32)


def _sc_body(qk, px, py, bmx, bmy, xt, yt, out,
             xcol, ycol, ftx, fty, tbl, bmbuf, buf0, buf1,
             isem0, isem1, osem0, osem1):
    wid = lax.axis_index("s") * 2 + lax.axis_index("c")
    b = wid // 16
    lrow0 = (wid % 16) * _RPW
    grow0 = b * _N + lrow0

    pltpu.sync_copy(px.at[b], xcol)
    pltpu.sync_copy(py.at[b], ycol)

    def _build(ft):
        def body(c, carry):
            s = pl.ds(c * 16, 16)
            v = plsc.load_gather(tbl, [bmbuf[s]])
            ft[s] = v * _SCALE
            return carry
        lax.fori_loop(0, _DPAD // 16, body, 0)

    pltpu.sync_copy(xt, tbl)
    pltpu.sync_copy(bmx, bmbuf)
    _build(ftx)
    pltpu.sync_copy(yt, tbl)
    pltpu.sync_copy(bmy, bmbuf)
    _build(fty)

    def _off(c, carry):
        s = pl.ds(c * 16, 16)
        xcol[s] = xcol[s] + (_P - 1)
        ycol[s] = ycol[s] + (_P - 1)
        return carry
    lax.fori_loop(0, _NCH, _off, 0)

    bufs = (buf0, buf1)
    isems = (isem0, isem1)
    osems = (osem0, osem1)

    def _in_copy(blk):
        return pltpu.make_async_copy(
            qk.at[pl.ds(grow0 + blk * _RB, _RB)], bufs[blk % 2], isems[blk % 2])

    def _out_copy(blk):
        return pltpu.make_async_copy(
            bufs[blk % 2], out.at[pl.ds(grow0 + blk * _RB, _RB)], osems[blk % 2])

    def _compute(buf, blk):
        def row_body(rp, carry):
            r0 = rp * 2
            lrow = lrow0 + blk * _RB + r0
            lv0 = jnp.broadcast_to(lrow, (16,)).astype(jnp.int32)
            lv1 = lv0 + 1
            xi0 = plsc.load_gather(xcol, [lv0]) - (_P - 1)
            yi0 = plsc.load_gather(ycol, [lv0]) - (_P - 1)
            xi1 = plsc.load_gather(xcol, [lv1]) - (_P - 1)
            yi1 = plsc.load_gather(ycol, [lv1]) - (_P - 1)
            def col_body(c, carry2):
                for u in range(4):
                    s = pl.ds((c * 4 + u) * 16, 16)
                    xc = xcol[s]
                    yc = ycol[s]
                    f0 = plsc.load_gather(ftx, [xc - xi0]) + plsc.load_gather(fty, [yc - yi0])
                    f1 = plsc.load_gather(ftx, [xc - xi1]) + plsc.load_gather(fty, [yc - yi1])
                    buf[r0, s] = buf[r0, s] + f0
                    buf[r0 + 1, s] = buf[r0 + 1, s] + f1
                return carry2
            return lax.fori_loop(0, _NCH // 4, col_body, carry)
        lax.fori_loop(0, _RB // 2, row_body, 0)

    _in_copy(0).start()
    for blk in range(_NBLK):
        _in_copy(blk).wait()
        if blk >= 1:
            _out_copy(blk - 1).wait()
        if blk + 1 < _NBLK:
            _in_copy(blk + 1).start()
        _compute(bufs[blk % 2], blk)
        _out_copy(blk).start()
    _out_copy(_NBLK - 1).wait()


_sc_bias = functools.partial(
    pl.kernel,
    mesh=plsc.VectorSubcoreMesh(core_axis_name="c", subcore_axis_name="s"),
    out_type=jax.ShapeDtypeStruct((_ROWS, _N), jnp.float32),
    scratch_types=[
        pltpu.VMEM((_N,), jnp.int32),
        pltpu.VMEM((_N,), jnp.int32),
        pltpu.VMEM((_DPAD,), jnp.float32),
        pltpu.VMEM((_DPAD,), jnp.float32),
        pltpu.VMEM((_NUM_BUCKETS,), jnp.float32),
        pltpu.VMEM((_DPAD,), jnp.int32),
        pltpu.VMEM((_RB, _N), jnp.float32),
        pltpu.VMEM((_RB, _N), jnp.float32),
        pltpu.SemaphoreType.DMA,
        pltpu.SemaphoreType.DMA,
        pltpu.SemaphoreType.DMA,
        pltpu.SemaphoreType.DMA,
    ],
    compiler_params=pltpu.CompilerParams(needs_layout_passes=False),
)(_sc_body)


def kernel(qk_dots, positions, x_table, y_table):
    b, n, _ = qk_dots.shape
    pos = positions.astype(jnp.int32)
    px = pos[:, :, 0]
    py = pos[:, :, 1]
    qk2 = qk_dots.reshape(b * n, n)
    out = _sc_bias(qk2, px, py,
                   jnp.asarray(_BMAP_X), jnp.asarray(_BMAP_Y),
                   x_table[:, 0], y_table[:, 0])
    return out.reshape(b, n, n)
